# Initial kernel scaffold; baseline (speedup 1.0000x reference)
import jax
import jax.numpy as jnp
from jax.experimental import pallas as pl


def kernel(extended_coord, extended_atype, nlist, mapping, type_table, W_e0, b_e0, W_node, W_edge):
    raise NotImplementedError("write your pallas kernel here")


# speedup vs baseline: 12.4076x; 12.4076x over previous
import functools

import jax
import jax.numpy as jnp
from jax import lax
from jax.experimental import pallas as pl
from jax.experimental.pallas import tpu as pltpu
from jax.experimental.pallas import tpu_sc as plsc

NTYPES = 8
NLOC = 10000
NNEI = 64
ND = 128
ED = 64
RCUT = 6.0
RCUT_SMTH = 5.0
EPS = 1e-6
CPAD = 16

E = NLOC * NNEI
BN = 80
BE = BN * NNEI
NB = NLOC // BN

RW = 80
NC, NS = 2, 16
NW = NC * NS
RPW = E // (NW * RW)


def _silu(x):
    return x / (1.0 + jnp.exp(-x))


def _dot(a, b):
    return jnp.dot(a, b, preferred_element_type=jnp.float32)


def _sc_gather(table, idx3):
    mesh = plsc.VectorSubcoreMesh(core_axis_name="c", subcore_axis_name="s")

    @functools.partial(
        pl.kernel,
        mesh=mesh,
        out_type=jax.ShapeDtypeStruct((E, ND), jnp.float32),
        scratch_types=[
            pltpu.VMEM((RPW, RW), jnp.int32),
            pltpu.VMEM((RW, ND), jnp.float32),
            pltpu.SemaphoreType.DMA,
        ],
    )
    def gk(table_hbm, idx_hbm, out_hbm, idx_v, row_v, sem):
        wid = lax.axis_index("s") * NC + lax.axis_index("c")
        base = wid * RPW
        pltpu.sync_copy(idx_hbm.at[wid], idx_v)

        def body(j, carry):
            pltpu.async_copy(table_hbm.at[idx_v.at[j]], row_v, sem).wait()
            pltpu.sync_copy(row_v, out_hbm.at[pl.ds((base + j) * RW, RW)])
            return carry

        lax.fori_loop(0, RPW, body, 0)

    return gk(table, idx3)


def _stage0_body(at_ref, tt_ref, w2_ref, ne_ref, p_ref):
    a = at_ref[:]
    ne = jnp.zeros((BN, ND), jnp.float32)
    for t in range(NTYPES):
        sel = (a == t).astype(jnp.float32)
        ne = ne + sel * tt_ref[t:t + 1, :]
    ne_ref[:] = ne
    p_ref[:] = _dot(ne, w2_ref[:])


def _stage0(at2, type_table, w2):
    return pl.pallas_call(
        _stage0_body,
        grid=(NB,),
        in_specs=[
            pl.BlockSpec((BN, 1), lambda i: (i, 0)),
            pl.BlockSpec((NTYPES, ND), lambda i: (0, 0)),
            pl.BlockSpec((ND, ED), lambda i: (0, 0)),
        ],
        out_specs=[
            pl.BlockSpec((BN, ND), lambda i: (i, 0)),
            pl.BlockSpec((BN, ED), lambda i: (i, 0)),
        ],
        out_shape=[
            jax.ShapeDtypeStruct((NLOC, ND), jnp.float32),
            jax.ShapeDtypeStruct((NLOC, ED), jnp.float32),
        ],
    )(at2, type_table, w2)


def _geom_body(t_ref, cc_ref, we_ref, be_ref, e_ref, sw_ref):
    cnb = t_ref[:, ED:ED + CPAD]
    cc = cc_ref[:]
    ccb = jnp.broadcast_to(cc[:, None, :], (BN, NNEI, CPAD)).reshape(BE, CPAD)
    lane = lax.broadcasted_iota(jnp.int32, (BE, CPAD), 1)
    diff = (cnb - ccb) + jnp.where(lane < 3, EPS, 0.0)
    dist = jnp.sqrt(jnp.sum(diff * diff, axis=1, keepdims=True))
    uu = jnp.clip((dist - RCUT_SMTH) / (RCUT - RCUT_SMTH), 0.0, 1.0)
    sw = uu * uu * uu * (-6.0 * uu * uu + 15.0 * uu - 10.0) + 1.0
    e_ref[:] = _silu(dist * we_ref[:] + be_ref[:]) * sw
    sw_ref[:] = sw


def _geom(t0g, cpad, we0, be0):
    return pl.pallas_call(
        _geom_body,
        grid=(NB,),
        in_specs=[
            pl.BlockSpec((BE, ND), lambda i: (i, 0)),
            pl.BlockSpec((BN, CPAD), lambda i: (i, 0)),
            pl.BlockSpec((1, ED), lambda i: (0, 0)),
            pl.BlockSpec((1, ED), lambda i: (0, 0)),
        ],
        out_specs=[
            pl.BlockSpec((BE, ED), lambda i: (i, 0)),
            pl.BlockSpec((BE, 1), lambda i: (i, 0)),
        ],
        out_shape=[
            jax.ShapeDtypeStruct((E, ED), jnp.float32),
            jax.ShapeDtypeStruct((E, 1), jnp.float32),
        ],
    )(t0g, cpad, we0, be0)


def _edge_body(last, g_direct, e_ref, nb_ref, n_ref, sw_ref, w1_ref, w2_ref,
               w3_ref, *out_refs):
    e = e_ref[:]
    sw = sw_ref[:]
    if g_direct:
        pre = _dot(e, w3_ref[:]) + nb_ref[:, :ED]
    else:
        pre = _dot(e, w3_ref[:]) + _dot(nb_ref[:], w2_ref[:])
    a = _dot(n_ref[:], w1_ref[:])
    sw3 = sw.reshape(BN, NNEI, 1)
    pre3 = pre.reshape(BN, NNEI, ED) + a[:, None, :]
    e3 = e.reshape(BN, NNEI, ED) + _silu(pre3) * sw3
    if last:
        (msg_ref,) = out_refs
    else:
        eo_ref, msg_ref = out_refs
        eo_ref[:] = e3.reshape(BE, ED)
    msg_ref[:] = jnp.sum(e3 * sw3, axis=1) * (1.0 / NNEI)


def _edge(e, nbg, node, sw, w1, w2, w3, last, g_direct=False):
    out_specs = [pl.BlockSpec((BN, ED), lambda i: (i, 0))]
    out_shape = [jax.ShapeDtypeStruct((NLOC, ED), jnp.float32)]
    if not last:
        out_specs.insert(0, pl.BlockSpec((BE, ED), lambda i: (i, 0)))
        out_shape.insert(0, jax.ShapeDtypeStruct((E, ED), jnp.float32))
    nb_spec = pl.BlockSpec((BE, ND), lambda i: (i, 0))
    return pl.pallas_call(
        functools.partial(_edge_body, last, g_direct),
        grid=(NB,),
        in_specs=[
            pl.BlockSpec((BE, ED), lambda i: (i, 0)),
            nb_spec,
            pl.BlockSpec((BN, ND), lambda i: (i, 0)),
            pl.BlockSpec((BE, 1), lambda i: (i, 0)),
            pl.BlockSpec((ND, ED), lambda i: (0, 0)),
            pl.BlockSpec((ND, ED), lambda i: (0, 0)),
            pl.BlockSpec((ED, ED), lambda i: (0, 0)),
        ],
        out_specs=out_specs,
        out_shape=out_shape,
    )(e, nbg, node, sw, w1, w2, w3)


def _node_body(n_ref, m_ref, wn1_ref, wn2_ref, no_ref):
    n = n_ref[:]
    h = _dot(n, wn1_ref[:]) + _dot(m_ref[:], wn2_ref[:])
    no_ref[:] = n + _silu(h)


def _node(node, msg, wn1, wn2):
    return pl.pallas_call(
        _node_body,
        grid=(NB,),
        in_specs=[
            pl.BlockSpec((BN, ND), lambda i: (i, 0)),
            pl.BlockSpec((BN, ED), lambda i: (i, 0)),
            pl.BlockSpec((ND, ND), lambda i: (0, 0)),
            pl.BlockSpec((ED, ND), lambda i: (0, 0)),
        ],
        out_specs=pl.BlockSpec((BN, ND), lambda i: (i, 0)),
        out_shape=jax.ShapeDtypeStruct((NLOC, ND), jnp.float32),
    )(node, msg, wn1, wn2)


def kernel(extended_coord, extended_atype, nlist, mapping, type_table,
           W_e0, b_e0, W_node, W_edge):
    coords = extended_coord[0].astype(jnp.float32)
    cpad = jnp.concatenate(
        [coords, jnp.zeros((NLOC, CPAD - 3), jnp.float32)], axis=1)
    at2 = extended_atype[0].astype(jnp.int32).reshape(NLOC, 1)
    idx3 = nlist[0].astype(jnp.int32).reshape(NW, RPW, RW)
    W1 = W_edge[:, :ND, :]
    W2 = W_edge[:, ND:2 * ND, :]
    W3 = W_edge[:, 2 * ND:, :]
    Wn1 = W_node[:, :ND, :]
    Wn2 = W_node[:, ND:, :]
    we0 = W_e0.reshape(1, ED)
    be0 = b_e0.reshape(1, ED)

    node0, p0 = _stage0(at2, type_table, W2[0])
    t0 = jnp.concatenate(
        [p0, cpad, jnp.zeros((NLOC, ND - ED - CPAD), jnp.float32)], axis=1)
    t0g = _sc_gather(t0, idx3)
    e0, sw = _geom(t0g, cpad, we0, be0)
    e1, msg0 = _edge(e0, t0g, node0, sw, W1[0], W2[0], W3[0],
                     last=False, g_direct=True)
    node1 = _node(node0, msg0, Wn1[0], Wn2[0])
    nb1 = _sc_gather(node1, idx3)
    e2, msg1 = _edge(e1, nb1, node1, sw, W1[1], W2[1], W3[1], last=False)
    node2 = _node(node1, msg1, Wn1[1], Wn2[1])
    nb2 = _sc_gather(node2, idx3)
    (msg2,) = _edge(e2, nb2, node2, sw, W1[2], W2[2], W3[2], last=True)
    node3 = _node(node2, msg2, Wn1[2], Wn2[2])
    return node3[None]
